# baseline (device time: 42419 ns/iter reference)
import jax
import jax.numpy as jnp
from jax import lax
from jax.experimental import pallas as pl
from jax.experimental.pallas import tpu as pltpu

N_DEV = 8
B_LOC = 2
SQ = 128
SKV = 128
HQ = 32
H_LOC = HQ // N_DEV
DH = 64
D_MODEL = 512
SCALE = 0.125
HB = 64

BF = jnp.bfloat16


def kernel(x, Wq, K_ext, V_ext, Wo):
    me = lax.axis_index("i")

    x2d = x.reshape(B_LOC * SQ, D_MODEL)
    K_loc = lax.dynamic_slice_in_dim(K_ext, me * B_LOC, B_LOC, axis=0)
    V_loc = lax.dynamic_slice_in_dim(V_ext, me * B_LOC, B_LOC, axis=0)
    K_r = K_loc.astype(BF).transpose(2, 0, 1, 3).reshape(HQ * B_LOC, SKV, DH)
    V_r = V_loc.astype(BF).transpose(2, 0, 1, 3).reshape(HQ * B_LOC, SKV, DH)

    blk = H_LOC * DH

    def body(x_ref, wq_ref, k_ref, v_ref, wo_ref, out_ref,
             x_bf, wq_all, wo_all, ctx_all,
             wq_send, wq_recv, wo_send, wo_recv):
        my = lax.axis_index("i")

        p4 = lax.rem(my, 4)
        zb = my // 4
        z4 = my - p4
        o_x = z4 + (p4 + 1 - 2 * lax.rem(p4, 2))
        o_y = z4 + (3 - p4)
        pxy = (3 - p4) + 1 - 2 * lax.rem(3 - p4, 2)
        o_xy = z4 + pxy
        o_z = lax.rem(my + 4, N_DEV)
        o_xz = lax.rem(o_x + 4, N_DEV)
        o_yz = lax.rem(o_y + 4, N_DEV)
        o_xyz = lax.rem(o_xy + 4, N_DEV)

        barrier_sem = pltpu.get_barrier_semaphore()
        for nbr in (o_x, o_y, o_z):
            pl.semaphore_signal(
                barrier_sem, inc=1,
                device_id=(nbr,), device_id_type=pl.DeviceIdType.MESH,
            )
        pl.semaphore_wait(barrier_sem, 3)

        x_bf[:, :] = x_ref[:, :].astype(BF)
        wq_all[my, :, :] = wq_ref[:, :].astype(BF)
        wo_all[my, :, :] = wo_ref[:, :].astype(BF)

        def attention(origin):
            q_all = jnp.dot(x_bf[:, :], wq_all[origin],
                            preferred_element_type=jnp.float32)
            q_all = q_all.astype(BF)
            for b in range(B_LOC):
                for hl in range(H_LOC):
                    head = (origin * H_LOC + hl) * B_LOC + b
                    q = q_all[b * SQ:(b + 1) * SQ, hl * DH:(hl + 1) * DH]
                    k = k_ref[head]
                    v = v_ref[head]
                    r0 = b * SQ
                    for (qs, kn) in ((0, HB), (HB, SKV)):
                        s = lax.dot_general(
                            q[qs:qs + HB], k[:kn],
                            (((1,), (1,)), ((), ())),
                            preferred_element_type=jnp.float32) * SCALE
                        m = jnp.max(s, axis=1, keepdims=True)
                        e = jnp.exp(s - m)
                        r = 1.0 / jnp.sum(e, axis=1, keepdims=True)
                        c = jnp.dot(e.astype(BF), v[:kn],
                                    preferred_element_type=jnp.float32) * r
                        ctx_all[origin, r0 + qs:r0 + qs + HB,
                                hl * DH:(hl + 1) * DH] = c.astype(BF)

        def contrib(origin, is_first=False):
            c = jnp.dot(ctx_all[origin], wo_all[origin],
                        preferred_element_type=jnp.float32)
            if is_first:
                out_ref[:, :] = c
            else:
                out_ref[:, :] += c

        pending = []

        def send(buf_all, slot, sem_arr, sem_idx, recv_sem_arr, target):
            rd = pltpu.make_async_remote_copy(
                src_ref=buf_all.at[slot],
                dst_ref=buf_all.at[slot],
                send_sem=sem_arr.at[sem_idx],
                recv_sem=recv_sem_arr.at[slot],
                device_id=(target,), device_id_type=pl.DeviceIdType.MESH,
            )
            rd.start()
            pending.append(rd)

        def wait_recv(buf_all, slot, recv_sem_arr):
            rd = pltpu.make_async_remote_copy(
                src_ref=buf_all.at[slot],
                dst_ref=buf_all.at[slot],
                send_sem=wq_send.at[0],
                recv_sem=recv_sem_arr.at[slot],
                device_id=(my,), device_id_type=pl.DeviceIdType.MESH,
            )
            rd.wait_recv()

        send(wq_all, my, wq_send, 0, wq_recv, o_x)
        send(wq_all, my, wq_send, 1, wq_recv, o_y)
        send(wq_all, my, wq_send, 2, wq_recv, o_z)
        send(wo_all, my, wo_send, 0, wo_recv, o_z)
        send(wo_all, my, wo_send, 1, wo_recv, o_y)
        send(wo_all, my, wo_send, 2, wo_recv, o_x)

        attention(my)
        contrib(my, is_first=True)

        wait_recv(wq_all, o_x, wq_recv)
        send(wq_all, o_x, wq_send, 3, wq_recv, o_y)
        send(wq_all, o_x, wq_send, 4, wq_recv, o_z)
        wait_recv(wo_all, o_z, wo_recv)
        send(wo_all, o_z, wo_send, 3, wo_recv, o_y)
        send(wo_all, o_z, wo_send, 4, wo_recv, o_x)
        attention(o_x)

        wait_recv(wq_all, o_y, wq_recv)
        send(wq_all, o_y, wq_send, 5, wq_recv, o_z)
        wait_recv(wo_all, o_y, wo_recv)
        send(wo_all, o_y, wo_send, 5, wo_recv, o_x)
        attention(o_y)
        contrib(o_y)

        wait_recv(wq_all, o_xy, wq_recv)
        send(wq_all, o_xy, wq_send, 6, wq_recv, o_z)
        wait_recv(wo_all, o_yz, wo_recv)
        send(wo_all, o_yz, wo_send, 6, wo_recv, o_x)
        attention(o_xy)

        wait_recv(wq_all, o_z, wq_recv)
        attention(o_z)
        contrib(o_z)
        wait_recv(wq_all, o_xz, wq_recv)
        attention(o_xz)
        wait_recv(wq_all, o_yz, wq_recv)
        attention(o_yz)
        contrib(o_yz)
        wait_recv(wq_all, o_xyz, wq_recv)
        attention(o_xyz)

        wait_recv(wo_all, o_x, wo_recv)
        contrib(o_x)
        wait_recv(wo_all, o_xz, wo_recv)
        contrib(o_xz)
        wait_recv(wo_all, o_xy, wo_recv)
        contrib(o_xy)
        wait_recv(wo_all, o_xyz, wo_recv)
        contrib(o_xyz)

        for rd in pending:
            rd.wait_send()

    out2d = pl.pallas_call(
        body,
        out_shape=jax.ShapeDtypeStruct((B_LOC * SQ, D_MODEL), jnp.float32),
        in_specs=[pl.BlockSpec(memory_space=pltpu.VMEM)] * 5,
        out_specs=pl.BlockSpec(memory_space=pltpu.VMEM),
        scratch_shapes=[
            pltpu.VMEM((B_LOC * SQ, D_MODEL), BF),
            pltpu.VMEM((N_DEV, D_MODEL, blk), BF),
            pltpu.VMEM((N_DEV, blk, D_MODEL), BF),
            pltpu.VMEM((N_DEV, B_LOC * SQ, blk), BF),
            pltpu.SemaphoreType.DMA((7,)),
            pltpu.SemaphoreType.DMA((N_DEV,)),
            pltpu.SemaphoreType.DMA((7,)),
            pltpu.SemaphoreType.DMA((N_DEV,)),
        ],
        compiler_params=pltpu.CompilerParams(collective_id=0),
    )(x2d, Wq, K_r, V_r, Wo)

    return out2d.reshape(B_LOC, SQ, D_MODEL)


# device time: 31553 ns/iter; 1.3444x vs baseline; 1.3444x over previous
import jax
import jax.numpy as jnp
from jax import lax
from jax.experimental import pallas as pl
from jax.experimental.pallas import tpu as pltpu

N_DEV = 8
B_LOC = 2
SQ = 128
SKV = 128
HQ = 32
H_LOC = HQ // N_DEV
DH = 64
D_MODEL = 512
SCALE = 0.125
HB = 64

BF = jnp.bfloat16


def kernel(x, Wq, K_ext, V_ext, Wo):
    me = lax.axis_index("i")

    x2d = x.reshape(B_LOC * SQ, D_MODEL)
    K_loc = lax.dynamic_slice_in_dim(K_ext, me * B_LOC, B_LOC, axis=0)
    V_loc = lax.dynamic_slice_in_dim(V_ext, me * B_LOC, B_LOC, axis=0)
    K_r = K_loc.astype(BF).transpose(2, 0, 1, 3).reshape(HQ * B_LOC, SKV, DH)
    V_r = V_loc.astype(BF).transpose(2, 0, 1, 3).reshape(HQ * B_LOC, SKV, DH)

    blk = H_LOC * DH

    def body(x_ref, wq_ref, k_ref, v_ref, wo_ref, out_ref,
             x_bf, wq_all, wo_all, ctx_all,
             wq_send, wq_recv, wo_send, wo_recv):
        my = lax.axis_index("i")

        p4 = lax.rem(my, 4)
        zb = my // 4
        z4 = my - p4
        o_x = z4 + (p4 + 1 - 2 * lax.rem(p4, 2))
        o_y = z4 + (3 - p4)
        pxy = (3 - p4) + 1 - 2 * lax.rem(3 - p4, 2)
        o_xy = z4 + pxy
        o_z = lax.rem(my + 4, N_DEV)
        o_xz = lax.rem(o_x + 4, N_DEV)
        o_yz = lax.rem(o_y + 4, N_DEV)
        o_xyz = lax.rem(o_xy + 4, N_DEV)

        barrier_sem = pltpu.get_barrier_semaphore()
        for nbr in (o_x, o_y, o_z):
            pl.semaphore_signal(
                barrier_sem, inc=1,
                device_id=(nbr,), device_id_type=pl.DeviceIdType.MESH,
            )
        pl.semaphore_wait(barrier_sem, 3)

        x_bf[:, :] = x_ref[:, :].astype(BF)
        wq_all[my, :, :] = wq_ref[:, :].astype(BF)
        wo_all[my, :, :] = wo_ref[:, :].astype(BF)

        qb_i = lax.broadcasted_iota(jnp.int32, (SQ, SKV), 0) // HB
        kb_i = lax.broadcasted_iota(jnp.int32, (SQ, SKV), 1) // HB
        mask_f = ((qb_i == kb_i) | (kb_i == 0)
                  | (lax.rem(qb_i + kb_i, 3) == 0)).astype(jnp.float32)

        BH = [(b, hl) for b in range(B_LOC) for hl in range(H_LOC)]

        def attention(origin):
            q_all = jnp.dot(x_bf[:, :], wq_all[origin],
                            preferred_element_type=jnp.float32)
            q_all = q_all.astype(BF)
            ss = []
            for b, hl in BH:
                head = (origin * H_LOC + hl) * B_LOC + b
                q = q_all[b * SQ:(b + 1) * SQ, hl * DH:(hl + 1) * DH]
                ss.append(lax.dot_general(
                    q, k_ref[head], (((1,), (1,)), ((), ())),
                    preferred_element_type=jnp.float32))
            s_all = jnp.stack(ss) * SCALE
            m = jnp.max(s_all, axis=2, keepdims=True)
            e = jnp.exp(s_all - m) * mask_f[None]
            r = 1.0 / jnp.sum(e, axis=2, keepdims=True)
            e = (e * r).astype(BF)
            for j, (b, hl) in enumerate(BH):
                head = (origin * H_LOC + hl) * B_LOC + b
                c = jnp.dot(e[j], v_ref[head],
                            preferred_element_type=jnp.float32)
                ctx_all[origin, b * SQ:(b + 1) * SQ,
                        hl * DH:(hl + 1) * DH] = c.astype(BF)

        def contrib(origin, is_first=False):
            c = jnp.dot(ctx_all[origin], wo_all[origin],
                        preferred_element_type=jnp.float32)
            if is_first:
                out_ref[:, :] = c
            else:
                out_ref[:, :] += c

        pending = []

        def send(buf_all, slot, sem_arr, sem_idx, recv_sem_arr, target):
            rd = pltpu.make_async_remote_copy(
                src_ref=buf_all.at[slot],
                dst_ref=buf_all.at[slot],
                send_sem=sem_arr.at[sem_idx],
                recv_sem=recv_sem_arr.at[slot],
                device_id=(target,), device_id_type=pl.DeviceIdType.MESH,
            )
            rd.start()
            pending.append(rd)

        def wait_recv(buf_all, slot, recv_sem_arr):
            rd = pltpu.make_async_remote_copy(
                src_ref=buf_all.at[slot],
                dst_ref=buf_all.at[slot],
                send_sem=wq_send.at[0],
                recv_sem=recv_sem_arr.at[slot],
                device_id=(my,), device_id_type=pl.DeviceIdType.MESH,
            )
            rd.wait_recv()

        send(wq_all, my, wq_send, 0, wq_recv, o_x)
        send(wq_all, my, wq_send, 1, wq_recv, o_y)
        send(wq_all, my, wq_send, 2, wq_recv, o_z)
        send(wo_all, my, wo_send, 0, wo_recv, o_z)
        send(wo_all, my, wo_send, 1, wo_recv, o_y)
        send(wo_all, my, wo_send, 2, wo_recv, o_x)

        attention(my)
        contrib(my, is_first=True)

        wait_recv(wq_all, o_x, wq_recv)
        send(wq_all, o_x, wq_send, 3, wq_recv, o_y)
        send(wq_all, o_x, wq_send, 4, wq_recv, o_z)
        wait_recv(wo_all, o_z, wo_recv)
        send(wo_all, o_z, wo_send, 3, wo_recv, o_y)
        send(wo_all, o_z, wo_send, 4, wo_recv, o_x)
        attention(o_x)

        wait_recv(wq_all, o_y, wq_recv)
        send(wq_all, o_y, wq_send, 5, wq_recv, o_z)
        wait_recv(wo_all, o_y, wo_recv)
        send(wo_all, o_y, wo_send, 5, wo_recv, o_x)
        attention(o_y)
        contrib(o_y)

        wait_recv(wq_all, o_xy, wq_recv)
        send(wq_all, o_xy, wq_send, 6, wq_recv, o_z)
        wait_recv(wo_all, o_yz, wo_recv)
        send(wo_all, o_yz, wo_send, 6, wo_recv, o_x)
        attention(o_xy)

        wait_recv(wq_all, o_z, wq_recv)
        attention(o_z)
        contrib(o_z)
        wait_recv(wq_all, o_xz, wq_recv)
        attention(o_xz)
        wait_recv(wq_all, o_yz, wq_recv)
        attention(o_yz)
        contrib(o_yz)
        wait_recv(wq_all, o_xyz, wq_recv)
        attention(o_xyz)

        wait_recv(wo_all, o_x, wo_recv)
        contrib(o_x)
        wait_recv(wo_all, o_xz, wo_recv)
        contrib(o_xz)
        wait_recv(wo_all, o_xy, wo_recv)
        contrib(o_xy)
        wait_recv(wo_all, o_xyz, wo_recv)
        contrib(o_xyz)

        for rd in pending:
            rd.wait_send()

    out2d = pl.pallas_call(
        body,
        out_shape=jax.ShapeDtypeStruct((B_LOC * SQ, D_MODEL), jnp.float32),
        in_specs=[pl.BlockSpec(memory_space=pltpu.VMEM)] * 5,
        out_specs=pl.BlockSpec(memory_space=pltpu.VMEM),
        scratch_shapes=[
            pltpu.VMEM((B_LOC * SQ, D_MODEL), BF),
            pltpu.VMEM((N_DEV, D_MODEL, blk), BF),
            pltpu.VMEM((N_DEV, blk, D_MODEL), BF),
            pltpu.VMEM((N_DEV, B_LOC * SQ, blk), BF),
            pltpu.SemaphoreType.DMA((7,)),
            pltpu.SemaphoreType.DMA((N_DEV,)),
            pltpu.SemaphoreType.DMA((7,)),
            pltpu.SemaphoreType.DMA((N_DEV,)),
        ],
        compiler_params=pltpu.CompilerParams(collective_id=0),
    )(x2d, Wq, K_r, V_r, Wo)

    return out2d.reshape(B_LOC, SQ, D_MODEL)


# device time: 31026 ns/iter; 1.3672x vs baseline; 1.0170x over previous
import jax
import jax.numpy as jnp
from jax import lax
from jax.experimental import pallas as pl
from jax.experimental.pallas import tpu as pltpu

N_DEV = 8
B_LOC = 2
SQ = 128
SKV = 128
HQ = 32
H_LOC = HQ // N_DEV
DH = 64
D_MODEL = 512
SCALE = 0.125
HB = 64

BF = jnp.bfloat16


def kernel(x, Wq, K_ext, V_ext, Wo):
    me = lax.axis_index("i")

    x2d = x.reshape(B_LOC * SQ, D_MODEL)
    K_loc = lax.dynamic_slice_in_dim(K_ext, me * B_LOC, B_LOC, axis=0)
    V_loc = lax.dynamic_slice_in_dim(V_ext, me * B_LOC, B_LOC, axis=0)
    K_r = K_loc.astype(BF).transpose(2, 0, 1, 3).reshape(HQ * B_LOC, SKV, DH)
    V_r = V_loc.astype(BF).transpose(2, 0, 1, 3).reshape(HQ * B_LOC, SKV, DH)

    blk = H_LOC * DH

    def body(x_ref, wq_ref, k_ref, v_ref, wo_ref, out_ref,
             x_bf, wq_all, wo_all, ctx_all,
             wq_send, wq_recv, wo_send, wo_recv):
        my = lax.axis_index("i")

        p4 = lax.rem(my, 4)
        zb = my // 4
        z4 = my - p4
        o_x = z4 + (p4 + 1 - 2 * lax.rem(p4, 2))
        o_y = z4 + (3 - p4)
        pxy = (3 - p4) + 1 - 2 * lax.rem(3 - p4, 2)
        o_xy = z4 + pxy
        o_z = lax.rem(my + 4, N_DEV)
        o_xz = lax.rem(o_x + 4, N_DEV)
        o_yz = lax.rem(o_y + 4, N_DEV)
        o_xyz = lax.rem(o_xy + 4, N_DEV)

        barrier_sem = pltpu.get_barrier_semaphore()
        for nbr in (o_x, o_y, o_z):
            pl.semaphore_signal(
                barrier_sem, inc=1,
                device_id=(nbr,), device_id_type=pl.DeviceIdType.MESH,
            )
        pl.semaphore_wait(barrier_sem, 3)

        x_bf[:, :] = x_ref[:, :].astype(BF)
        wq_all[my, :, :] = wq_ref[:, :].astype(BF)
        wo_all[my, :, :] = wo_ref[:, :].astype(BF)

        qb_i = lax.broadcasted_iota(jnp.int32, (SQ, SKV), 0) // HB
        kb_i = lax.broadcasted_iota(jnp.int32, (SQ, SKV), 1) // HB
        mask_f = ((qb_i == kb_i) | (kb_i == 0)
                  | (lax.rem(qb_i + kb_i, 3) == 0)).astype(jnp.float32)

        BH = [(b, hl) for b in range(B_LOC) for hl in range(H_LOC)]

        def attention(origin):
            q_all = jnp.dot(x_bf[:, :], wq_all[origin],
                            preferred_element_type=jnp.float32)
            q_all = q_all.astype(BF)
            ss = []
            for b, hl in BH:
                head = (origin * H_LOC + hl) * B_LOC + b
                q = q_all[b * SQ:(b + 1) * SQ, hl * DH:(hl + 1) * DH]
                ss.append(lax.dot_general(
                    q, k_ref[head], (((1,), (1,)), ((), ())),
                    preferred_element_type=jnp.float32))
            s_all = jnp.stack(ss) * SCALE
            e = jnp.exp(s_all) * mask_f[None]
            r = 1.0 / jnp.sum(e, axis=2, keepdims=True)
            e_bf = e.astype(BF)
            for j, (b, hl) in enumerate(BH):
                head = (origin * H_LOC + hl) * B_LOC + b
                c = jnp.dot(e_bf[j], v_ref[head],
                            preferred_element_type=jnp.float32) * r[j]
                ctx_all[origin, b * SQ:(b + 1) * SQ,
                        hl * DH:(hl + 1) * DH] = c.astype(BF)

        def contrib(origin, is_first=False):
            c = jnp.dot(ctx_all[origin], wo_all[origin],
                        preferred_element_type=jnp.float32)
            if is_first:
                out_ref[:, :] = c
            else:
                out_ref[:, :] += c

        pending = []

        def send(buf_all, slot, sem_arr, sem_idx, recv_sem_arr, target):
            rd = pltpu.make_async_remote_copy(
                src_ref=buf_all.at[slot],
                dst_ref=buf_all.at[slot],
                send_sem=sem_arr.at[sem_idx],
                recv_sem=recv_sem_arr.at[slot],
                device_id=(target,), device_id_type=pl.DeviceIdType.MESH,
            )
            rd.start()
            pending.append(rd)

        def wait_recv(buf_all, slot, recv_sem_arr):
            rd = pltpu.make_async_remote_copy(
                src_ref=buf_all.at[slot],
                dst_ref=buf_all.at[slot],
                send_sem=wq_send.at[0],
                recv_sem=recv_sem_arr.at[slot],
                device_id=(my,), device_id_type=pl.DeviceIdType.MESH,
            )
            rd.wait_recv()

        send(wq_all, my, wq_send, 0, wq_recv, o_x)
        send(wq_all, my, wq_send, 1, wq_recv, o_y)
        send(wq_all, my, wq_send, 2, wq_recv, o_z)
        send(wo_all, my, wo_send, 0, wo_recv, o_z)
        send(wo_all, my, wo_send, 1, wo_recv, o_y)
        send(wo_all, my, wo_send, 2, wo_recv, o_x)

        attention(my)
        contrib(my, is_first=True)

        wait_recv(wq_all, o_x, wq_recv)
        send(wq_all, o_x, wq_send, 3, wq_recv, o_y)
        send(wq_all, o_x, wq_send, 4, wq_recv, o_z)
        wait_recv(wo_all, o_z, wo_recv)
        send(wo_all, o_z, wo_send, 3, wo_recv, o_y)
        send(wo_all, o_z, wo_send, 4, wo_recv, o_x)
        attention(o_x)

        wait_recv(wq_all, o_y, wq_recv)
        send(wq_all, o_y, wq_send, 5, wq_recv, o_z)
        wait_recv(wo_all, o_y, wo_recv)
        send(wo_all, o_y, wo_send, 5, wo_recv, o_x)
        attention(o_y)
        contrib(o_y)

        wait_recv(wq_all, o_xy, wq_recv)
        send(wq_all, o_xy, wq_send, 6, wq_recv, o_z)
        wait_recv(wo_all, o_yz, wo_recv)
        send(wo_all, o_yz, wo_send, 6, wo_recv, o_x)
        attention(o_xy)

        wait_recv(wq_all, o_z, wq_recv)
        attention(o_z)
        contrib(o_z)
        wait_recv(wq_all, o_xz, wq_recv)
        attention(o_xz)
        wait_recv(wq_all, o_yz, wq_recv)
        attention(o_yz)
        contrib(o_yz)
        wait_recv(wq_all, o_xyz, wq_recv)
        attention(o_xyz)

        wait_recv(wo_all, o_x, wo_recv)
        contrib(o_x)
        wait_recv(wo_all, o_xz, wo_recv)
        contrib(o_xz)
        wait_recv(wo_all, o_xy, wo_recv)
        contrib(o_xy)
        wait_recv(wo_all, o_xyz, wo_recv)
        contrib(o_xyz)

        for rd in pending:
            rd.wait_send()

    out2d = pl.pallas_call(
        body,
        out_shape=jax.ShapeDtypeStruct((B_LOC * SQ, D_MODEL), jnp.float32),
        in_specs=[pl.BlockSpec(memory_space=pltpu.VMEM)] * 5,
        out_specs=pl.BlockSpec(memory_space=pltpu.VMEM),
        scratch_shapes=[
            pltpu.VMEM((B_LOC * SQ, D_MODEL), BF),
            pltpu.VMEM((N_DEV, D_MODEL, blk), BF),
            pltpu.VMEM((N_DEV, blk, D_MODEL), BF),
            pltpu.VMEM((N_DEV, B_LOC * SQ, blk), BF),
            pltpu.SemaphoreType.DMA((7,)),
            pltpu.SemaphoreType.DMA((N_DEV,)),
            pltpu.SemaphoreType.DMA((7,)),
            pltpu.SemaphoreType.DMA((N_DEV,)),
        ],
        compiler_params=pltpu.CompilerParams(collective_id=0),
    )(x2d, Wq, K_r, V_r, Wo)

    return out2d.reshape(B_LOC, SQ, D_MODEL)


# device time: 29944 ns/iter; 1.4166x vs baseline; 1.0361x over previous
import jax
import jax.numpy as jnp
from jax import lax
from jax.experimental import pallas as pl
from jax.experimental.pallas import tpu as pltpu

N_DEV = 8
B_LOC = 2
SQ = 128
SKV = 128
HQ = 32
H_LOC = HQ // N_DEV
DH = 64
D_MODEL = 512
SCALE = 0.125
HB = 64

BF = jnp.bfloat16


def kernel(x, Wq, K_ext, V_ext, Wo):
    me = lax.axis_index("i")

    x2d = x.reshape(B_LOC * SQ, D_MODEL)
    K_loc = lax.dynamic_slice_in_dim(K_ext, me * B_LOC, B_LOC, axis=0)
    V_loc = lax.dynamic_slice_in_dim(V_ext, me * B_LOC, B_LOC, axis=0)
    K_r = K_loc.reshape(B_LOC, SKV, HQ * DH).astype(BF)
    V_r = V_loc.reshape(B_LOC, SKV, HQ * DH).astype(BF)

    blk = H_LOC * DH

    def body(x_ref, wq_ref, k_ref, v_ref, wo_ref, out_ref,
             x_bf, wq_all, wo_all, ctx_all,
             wq_send, wq_recv, wo_send, wo_recv):
        my = lax.axis_index("i")

        p4 = lax.rem(my, 4)
        zb = my // 4
        z4 = my - p4
        o_x = z4 + (p4 + 1 - 2 * lax.rem(p4, 2))
        o_y = z4 + (3 - p4)
        pxy = (3 - p4) + 1 - 2 * lax.rem(3 - p4, 2)
        o_xy = z4 + pxy
        o_z = lax.rem(my + 4, N_DEV)
        o_xz = lax.rem(o_x + 4, N_DEV)
        o_yz = lax.rem(o_y + 4, N_DEV)
        o_xyz = lax.rem(o_xy + 4, N_DEV)

        barrier_sem = pltpu.get_barrier_semaphore()
        for nbr in (o_x, o_y, o_z):
            pl.semaphore_signal(
                barrier_sem, inc=1,
                device_id=(nbr,), device_id_type=pl.DeviceIdType.MESH,
            )
        pl.semaphore_wait(barrier_sem, 3)

        x_bf[:, :] = x_ref[:, :].astype(BF)
        wq_all[my, :, :] = wq_ref[:, :].astype(BF)
        wo_all[my, :, :] = wo_ref[:, :].astype(BF)

        qb_i = lax.broadcasted_iota(jnp.int32, (SQ, SKV), 0) // HB
        kb_i = lax.broadcasted_iota(jnp.int32, (SQ, SKV), 1) // HB
        mask_f = ((qb_i == kb_i) | (kb_i == 0)
                  | (lax.rem(qb_i + kb_i, 3) == 0)).astype(jnp.float32)

        BH = [(b, hl) for b in range(B_LOC) for hl in range(H_LOC)]

        def attention(origin):
            q_all = jnp.dot(x_bf[:, :], wq_all[origin],
                            preferred_element_type=jnp.float32)
            q_all = q_all.astype(BF)
            kslab = [k_ref[b, :, pl.ds(origin * blk, blk)]
                     for b in range(B_LOC)]
            vslab = [v_ref[b, :, pl.ds(origin * blk, blk)]
                     for b in range(B_LOC)]
            ss = []
            for b, hl in BH:
                q = q_all[b * SQ:(b + 1) * SQ, hl * DH:(hl + 1) * DH]
                k = kslab[b][:, hl * DH:(hl + 1) * DH]
                ss.append(lax.dot_general(
                    q, k, (((1,), (1,)), ((), ())),
                    preferred_element_type=jnp.float32))
            s_all = jnp.stack(ss) * SCALE
            e = jnp.exp(s_all) * mask_f[None]
            r = 1.0 / jnp.sum(e, axis=2, keepdims=True)
            e_bf = e.astype(BF)
            for j, (b, hl) in enumerate(BH):
                v = vslab[b][:, hl * DH:(hl + 1) * DH]
                c = jnp.dot(e_bf[j], v,
                            preferred_element_type=jnp.float32) * r[j]
                ctx_all[origin, b * SQ:(b + 1) * SQ,
                        hl * DH:(hl + 1) * DH] = c.astype(BF)

        def contrib(origin, is_first=False):
            c = jnp.dot(ctx_all[origin], wo_all[origin],
                        preferred_element_type=jnp.float32)
            if is_first:
                out_ref[:, :] = c
            else:
                out_ref[:, :] += c

        pending = []

        def send(buf_all, slot, sem_arr, sem_idx, recv_sem_arr, target):
            rd = pltpu.make_async_remote_copy(
                src_ref=buf_all.at[slot],
                dst_ref=buf_all.at[slot],
                send_sem=sem_arr.at[sem_idx],
                recv_sem=recv_sem_arr.at[slot],
                device_id=(target,), device_id_type=pl.DeviceIdType.MESH,
            )
            rd.start()
            pending.append(rd)

        def wait_recv(buf_all, slot, recv_sem_arr):
            rd = pltpu.make_async_remote_copy(
                src_ref=buf_all.at[slot],
                dst_ref=buf_all.at[slot],
                send_sem=wq_send.at[0],
                recv_sem=recv_sem_arr.at[slot],
                device_id=(my,), device_id_type=pl.DeviceIdType.MESH,
            )
            rd.wait_recv()

        send(wq_all, my, wq_send, 0, wq_recv, o_x)
        send(wq_all, my, wq_send, 1, wq_recv, o_y)
        send(wq_all, my, wq_send, 2, wq_recv, o_z)
        send(wo_all, my, wo_send, 0, wo_recv, o_z)
        send(wo_all, my, wo_send, 1, wo_recv, o_y)
        send(wo_all, my, wo_send, 2, wo_recv, o_x)

        attention(my)
        contrib(my, is_first=True)

        wait_recv(wq_all, o_x, wq_recv)
        send(wq_all, o_x, wq_send, 3, wq_recv, o_y)
        send(wq_all, o_x, wq_send, 4, wq_recv, o_z)
        wait_recv(wo_all, o_z, wo_recv)
        send(wo_all, o_z, wo_send, 3, wo_recv, o_y)
        send(wo_all, o_z, wo_send, 4, wo_recv, o_x)
        attention(o_x)

        wait_recv(wq_all, o_y, wq_recv)
        send(wq_all, o_y, wq_send, 5, wq_recv, o_z)
        wait_recv(wo_all, o_y, wo_recv)
        send(wo_all, o_y, wo_send, 5, wo_recv, o_x)
        attention(o_y)
        contrib(o_y)

        wait_recv(wq_all, o_xy, wq_recv)
        send(wq_all, o_xy, wq_send, 6, wq_recv, o_z)
        wait_recv(wo_all, o_yz, wo_recv)
        send(wo_all, o_yz, wo_send, 6, wo_recv, o_x)
        attention(o_xy)

        wait_recv(wq_all, o_z, wq_recv)
        attention(o_z)
        contrib(o_z)
        wait_recv(wq_all, o_xz, wq_recv)
        attention(o_xz)
        wait_recv(wq_all, o_yz, wq_recv)
        attention(o_yz)
        contrib(o_yz)
        wait_recv(wq_all, o_xyz, wq_recv)
        attention(o_xyz)

        wait_recv(wo_all, o_x, wo_recv)
        contrib(o_x)
        wait_recv(wo_all, o_xz, wo_recv)
        contrib(o_xz)
        wait_recv(wo_all, o_xy, wo_recv)
        contrib(o_xy)
        wait_recv(wo_all, o_xyz, wo_recv)
        contrib(o_xyz)

        for rd in pending:
            rd.wait_send()

    out2d = pl.pallas_call(
        body,
        out_shape=jax.ShapeDtypeStruct((B_LOC * SQ, D_MODEL), jnp.float32),
        in_specs=[pl.BlockSpec(memory_space=pltpu.VMEM)] * 5,
        out_specs=pl.BlockSpec(memory_space=pltpu.VMEM),
        scratch_shapes=[
            pltpu.VMEM((B_LOC * SQ, D_MODEL), BF),
            pltpu.VMEM((N_DEV, D_MODEL, blk), BF),
            pltpu.VMEM((N_DEV, blk, D_MODEL), BF),
            pltpu.VMEM((N_DEV, B_LOC * SQ, blk), BF),
            pltpu.SemaphoreType.DMA((7,)),
            pltpu.SemaphoreType.DMA((N_DEV,)),
            pltpu.SemaphoreType.DMA((7,)),
            pltpu.SemaphoreType.DMA((N_DEV,)),
        ],
        compiler_params=pltpu.CompilerParams(collective_id=0),
    )(x2d, Wq, K_r, V_r, Wo)

    return out2d.reshape(B_LOC, SQ, D_MODEL)
